# baseline (speedup 1.0000x reference)
import functools

import jax
import jax.numpy as jnp
from jax import lax
from jax.experimental import pallas as pl
from jax.experimental.pallas import tpu as pltpu
from jax.experimental.pallas import tpu_sc as plsc

_NC = 2
_NS = 16
_L = 16
_NW = _NC * _NS


def _lse_body(logits_ref, out_ref):
    t = pl.program_id(1)
    x = logits_ref[0]
    s = jnp.sum(jnp.exp(x), axis=1, keepdims=True)
    partial = jnp.sum(jnp.log(s)).reshape(1, 1)

    @pl.when(t == 0)
    def _():
        out_ref[0] = jnp.zeros((1, 1), jnp.float32)

    out_ref[0] += partial


def _sc_gather_body(ppw, cpw, table_hbm, hyp_hbm, out_hbm,
                    hyp_v, buf_v, acc_v, sem):
    wid = lax.axis_index("s") * _NC + lax.axis_index("c")
    base = wid * ppw
    pltpu.sync_copy(hyp_hbm.at[pl.ds(base, ppw)], hyp_v)
    iota = lax.iota(jnp.int32, _L)

    def group(g, acc):
        g0 = pl.multiple_of(g * cpw, cpw)
        hvs = [hyp_v[pl.ds(g0 + j * _L, _L)] for j in range(cpw // _L)]
        copies = []
        for s in range(cpw):
            h = hvs[s // _L][s % _L]
            voff = pl.multiple_of(
                lax.shift_left(lax.shift_right_logical(h, 7), 7), 128)
            row0 = pl.multiple_of(base + g0 + (s & ~7), 8)
            copies.append(pltpu.async_copy(
                table_hbm.at[pl.ds(row0, 8), pl.ds(voff, 128)],
                buf_v.at[s], sem))
        for c in copies:
            c.wait()
        for s in range(cpw):
            h = hvs[s // _L][s % _L]
            lo = lax.shift_left(
                lax.shift_right_logical(jnp.bitwise_and(h, 127), 4), 4)
            chunk = buf_v[s, s % 8, pl.ds(lo, _L)]
            lane = jnp.bitwise_and(h, _L - 1)
            acc = acc + jnp.where(iota == lane, chunk, 0.0)
        return acc

    acc = lax.fori_loop(0, ppw // cpw, group, jnp.zeros((_L,), jnp.float32))
    acc_v[...] = acc
    pltpu.sync_copy(acc_v, out_hbm.at[wid])


def kernel(logits, hyp):
    b, t, v = logits.shape
    bt = b * t
    ppw = bt // _NW

    table = logits.reshape(bt, v)
    hyp_flat = hyp.astype(jnp.int32).reshape(bt)

    sc_gather = functools.partial(
        pl.kernel,
        out_type=jax.ShapeDtypeStruct((_NW, _L), jnp.float32),
        mesh=plsc.VectorSubcoreMesh(core_axis_name="c", subcore_axis_name="s"),
        scratch_types=[
            pltpu.VMEM((ppw,), jnp.int32),
            pltpu.VMEM((32, 8, 128), jnp.float32),
            pltpu.VMEM((_L,), jnp.float32),
            pltpu.SemaphoreType.DMA,
        ],
    )(functools.partial(_sc_gather_body, ppw, 32))
    g_parts = sc_gather(table, hyp_flat)
    g = g_parts.reshape(b, (_NW // b) * _L).sum(axis=1)

    tb = 256
    nt = t // tb
    lse = pl.pallas_call(
        _lse_body,
        grid=(b, nt),
        in_specs=[pl.BlockSpec((1, tb, v), lambda i, j: (i, j, 0))],
        out_specs=pl.BlockSpec((1, 1, 1), lambda i, j: (i, 0, 0)),
        out_shape=jax.ShapeDtypeStruct((b, 1, 1), jnp.float32),
        compiler_params=pltpu.CompilerParams(
            dimension_semantics=("arbitrary", "arbitrary"),
        ),
    )(logits)

    return g - lse[:, 0, 0]


# speedup vs baseline: 1.0114x; 1.0114x over previous
import functools

import jax
import jax.numpy as jnp
from jax import lax
from jax.experimental import pallas as pl
from jax.experimental.pallas import tpu as pltpu
from jax.experimental.pallas import tpu_sc as plsc

_NC = 1
_NS = 16
_L = 16
_NW = _NC * _NS


def _lse_body(logits_ref, out_ref):
    t = pl.program_id(1)
    x = logits_ref[0]
    s = jnp.sum(jnp.exp(x), axis=1, keepdims=True)
    partial = jnp.sum(jnp.log(s)).reshape(1, 1)

    @pl.when(t == 0)
    def _():
        out_ref[0] = jnp.zeros((1, 1), jnp.float32)

    out_ref[0] += partial


def _sc_gather_body(ppw, cpw, table_hbm, hyp_hbm, out_hbm,
                    hyp_v, buf_v, acc_v, sem):
    wid = lax.axis_index("s") * _NC + lax.axis_index("c")
    base = wid * ppw
    pltpu.sync_copy(hyp_hbm.at[pl.ds(base, ppw)], hyp_v)
    iota = lax.iota(jnp.int32, _L)

    def group(g, acc):
        g0 = pl.multiple_of(g * cpw, cpw)
        hvs = [hyp_v[pl.ds(g0 + j * _L, _L)] for j in range(cpw // _L)]
        copies = []
        for s in range(cpw):
            h = hvs[s // _L][s % _L]
            voff = pl.multiple_of(
                lax.shift_left(lax.shift_right_logical(h, 7), 7), 128)
            row0 = pl.multiple_of(base + g0 + (s & ~7), 8)
            copies.append(pltpu.async_copy(
                table_hbm.at[pl.ds(row0, 8), pl.ds(voff, 128)],
                buf_v.at[s], sem))
        for c in copies:
            c.wait()
        for s in range(cpw):
            h = hvs[s // _L][s % _L]
            lo = lax.shift_left(
                lax.shift_right_logical(jnp.bitwise_and(h, 127), 4), 4)
            chunk = buf_v[s, s % 8, pl.ds(lo, _L)]
            lane = jnp.bitwise_and(h, _L - 1)
            acc = acc + jnp.where(iota == lane, chunk, 0.0)
        return acc

    acc = lax.fori_loop(0, ppw // cpw, group, jnp.zeros((_L,), jnp.float32))
    acc_v[...] = acc
    pltpu.sync_copy(acc_v, out_hbm.at[wid])


def kernel(logits, hyp):
    b, t, v = logits.shape
    bt = b * t
    ppw = bt // _NW

    table = logits.reshape(bt, v)
    hyp_flat = hyp.astype(jnp.int32).reshape(bt)

    sc_gather = functools.partial(
        pl.kernel,
        out_type=jax.ShapeDtypeStruct((_NW, _L), jnp.float32),
        mesh=plsc.VectorSubcoreMesh(core_axis_name="c", subcore_axis_name="s", num_cores=1),
        scratch_types=[
            pltpu.VMEM((ppw,), jnp.int32),
            pltpu.VMEM((32, 8, 128), jnp.float32),
            pltpu.VMEM((_L,), jnp.float32),
            pltpu.SemaphoreType.DMA,
        ],
    )(functools.partial(_sc_gather_body, ppw, 32))
    g_parts = sc_gather(table, hyp_flat)
    g = g_parts.reshape(b, (_NW // b) * _L).sum(axis=1)

    tb = 256
    nt = t // tb
    lse = pl.pallas_call(
        _lse_body,
        grid=(b, nt),
        in_specs=[pl.BlockSpec((1, tb, v), lambda i, j: (i, j, 0))],
        out_specs=pl.BlockSpec((1, 1, 1), lambda i, j: (i, 0, 0)),
        out_shape=jax.ShapeDtypeStruct((b, 1, 1), jnp.float32),
        compiler_params=pltpu.CompilerParams(
            dimension_semantics=("arbitrary", "arbitrary"),
        ),
    )(logits)

    return g - lse[:, 0, 0]
